# baseline (speedup 1.0000x reference)
import functools

import jax
import jax.numpy as jnp
import numpy as np
from jax import lax
from jax.experimental import pallas as pl
from jax.experimental.pallas import tpu as pltpu
from jax.experimental.pallas import tpu_sc as plsc

VOCAB = 100000
D = 128
S = 200
B = 1024

NC = 2
NS = 16
NW = NC * NS
ROWS_PER_W = B // NW
IDS_PER_W = ROWS_PER_W * S
NBUF = 3


def _pos_enc() -> np.ndarray:
    pos = np.arange(S, dtype=np.float32)[:, None]
    i = np.arange(D, dtype=np.float32)[None, :]
    angle_rates = 1.0 / np.power(10000.0, (2.0 * np.floor(i / 2.0)) / np.float32(D))
    angles = pos * angle_rates
    pe = np.zeros((S, D), dtype=np.float32)
    pe[:, 0::2] = np.sin(angles[:, 0::2])
    pe[:, 1::2] = np.cos(angles[:, 1::2])
    return pe


_MESH = plsc.VectorSubcoreMesh(core_axis_name="c", subcore_axis_name="s")


@functools.partial(
    pl.kernel,
    out_type=jax.ShapeDtypeStruct((B, S, D), jnp.float32),
    mesh=_MESH,
    scratch_types=[
        pltpu.VMEM((IDS_PER_W,), jnp.int32),
        pltpu.VMEM((S, D), jnp.float32),
        [pltpu.VMEM((S, D), jnp.float32)] * NBUF,
        [pltpu.SemaphoreType.DMA] * NBUF,
        [pltpu.SemaphoreType.DMA] * NBUF,
    ],
)
def _emb(table_hbm, idx_hbm, pe_hbm, out_hbm, idx_v, pe_v, bufs, gsems, osems):
    wid = lax.axis_index("s") * NC + lax.axis_index("c")
    pltpu.sync_copy(idx_hbm.at[pl.ds(wid * IDS_PER_W, IDS_PER_W)], idx_v)
    pltpu.sync_copy(pe_hbm, pe_v)

    def gather_descs(b, k):
        return (
            pltpu.make_async_copy(
                table_hbm.at[idx_v.at[pl.ds(b * S, 128)]],
                bufs[k].at[pl.ds(0, 128)], gsems[k]),
            pltpu.make_async_copy(
                table_hbm.at[idx_v.at[pl.ds(b * S + 128, S - 128)]],
                bufs[k].at[pl.ds(128, S - 128)], gsems[k]),
        )

    def issue_gather(b, k):
        for cp in gather_descs(b, k):
            cp.start()

    def wait_gather(b, k):
        for cp in gather_descs(b, k):
            cp.wait()

    def out_desc(b, k):
        return pltpu.make_async_copy(
            bufs[k], out_hbm.at[wid * ROWS_PER_W + b], osems[k])

    def add_pe(k):
        buf = bufs[k]

        def add_rows(t, c2):
            for u in range(2):
                i = 2 * t + u
                for j in range(D // 16):
                    sl = pl.ds(j * 16, 16)
                    buf[i, sl] = buf[i, sl] + pe_v[i, sl]
            return c2

        lax.fori_loop(0, S // 2, add_rows, 0)

    issue_gather(0, 0)
    issue_gather(1, 1)

    def trio(p, carry):
        for j in range(NBUF):
            b = NBUF * p + j
            wait_gather(b, j)
            add_pe(j)
            kn = (j + NBUF - 1) % NBUF

            def recycle():
                out_desc(b - 1, kn).wait()
                issue_gather(b + 2, kn)

            if j == 0:
                @pl.when(p > 0)
                def _():
                    recycle()

                @pl.when(p == 0)
                def _():
                    issue_gather(b + 2, kn)
            else:
                recycle()
            out_desc(b, j).start()
        return carry

    lax.fori_loop(0, (ROWS_PER_W - 2) // NBUF, trio, 0)
    for b, k in ((ROWS_PER_W - 2, 0), (ROWS_PER_W - 1, 1)):
        wait_gather(b, k)
        add_pe(k)
        out_desc(b, k).start()
    for b, k in ((ROWS_PER_W - 3, 2), (ROWS_PER_W - 2, 0), (ROWS_PER_W - 1, 1)):
        out_desc(b, k).wait()


def kernel(inputs, table):
    idx_flat = inputs.reshape(-1).astype(jnp.int32)
    pe = jnp.asarray(_pos_enc())
    return _emb(table, idx_flat, pe)


# speedup vs baseline: 1.0237x; 1.0237x over previous
import functools

import jax
import jax.numpy as jnp
import numpy as np
from jax import lax
from jax.experimental import pallas as pl
from jax.experimental.pallas import tpu as pltpu
from jax.experimental.pallas import tpu_sc as plsc

VOCAB = 100000
D = 128
S = 200
B = 1024

NC = 2
NS = 16
NW = NC * NS
ROWS_PER_W = B // NW
IDS_PER_W = ROWS_PER_W * S
NBUF = 3


def _pos_enc() -> np.ndarray:
    pos = np.arange(S, dtype=np.float32)[:, None]
    i = np.arange(D, dtype=np.float32)[None, :]
    angle_rates = 1.0 / np.power(10000.0, (2.0 * np.floor(i / 2.0)) / np.float32(D))
    angles = pos * angle_rates
    pe = np.zeros((S, D), dtype=np.float32)
    pe[:, 0::2] = np.sin(angles[:, 0::2])
    pe[:, 1::2] = np.cos(angles[:, 1::2])
    return pe


_MESH = plsc.VectorSubcoreMesh(core_axis_name="c", subcore_axis_name="s")


@functools.partial(
    pl.kernel,
    out_type=jax.ShapeDtypeStruct((B, S, D), jnp.float32),
    mesh=_MESH,
    scratch_types=[
        pltpu.VMEM((IDS_PER_W,), jnp.int32),
        pltpu.VMEM((S, D), jnp.float32),
        [pltpu.VMEM((S, D), jnp.float32)] * NBUF,
        [pltpu.SemaphoreType.DMA] * NBUF,
        [pltpu.SemaphoreType.DMA] * NBUF,
    ],
)
def _emb(table_hbm, idx_hbm, pe_hbm, out_hbm, idx_v, pe_v, bufs, gsems, osems):
    wid = lax.axis_index("s") * NC + lax.axis_index("c")
    pltpu.sync_copy(idx_hbm.at[pl.ds(wid * IDS_PER_W, IDS_PER_W)], idx_v)
    pltpu.sync_copy(pe_hbm, pe_v)

    def _gparts(b, k):
        return (
            (table_hbm.at[idx_v.at[pl.ds(b * S, 128)]],
             bufs[k].at[pl.ds(0, 128)], gsems[k]),
            (table_hbm.at[idx_v.at[pl.ds(b * S + 128, S - 128)]],
             bufs[k].at[pl.ds(128, S - 128)], gsems[k]),
        )

    def issue_gather(b, k):
        prefill_pe(k)
        for src, dst, sem in _gparts(b, k):
            pltpu.async_copy(src, dst, sem, add=True)

    def wait_gather(b, k):
        for src, dst, sem in _gparts(b, k):
            pltpu.make_async_copy(src, dst, sem).wait()

    def prefill_pe(k):
        buf = bufs[k]

        def cp_rows(t, c2):
            for u in range(2):
                i = 2 * t + u
                for j in range(D // 16):
                    sl = pl.ds(j * 16, 16)
                    buf[i, sl] = pe_v[i, sl]
            return c2

        lax.fori_loop(0, S // 2, cp_rows, 0)

    def out_desc(b, k):
        return pltpu.make_async_copy(
            bufs[k], out_hbm.at[wid * ROWS_PER_W + b], osems[k])

    issue_gather(0, 0)
    issue_gather(1, 1)

    def trio(p, carry):
        for j in range(NBUF):
            b = NBUF * p + j
            wait_gather(b, j)
            kn = (j + NBUF - 1) % NBUF

            def recycle():
                out_desc(b - 1, kn).wait()
                issue_gather(b + 2, kn)

            if j == 0:
                @pl.when(p > 0)
                def _():
                    recycle()

                @pl.when(p == 0)
                def _():
                    issue_gather(b + 2, kn)
            else:
                recycle()
            out_desc(b, j).start()
        return carry

    lax.fori_loop(0, (ROWS_PER_W - 2) // NBUF, trio, 0)
    for b, k in ((ROWS_PER_W - 2, 0), (ROWS_PER_W - 1, 1)):
        wait_gather(b, k)
        out_desc(b, k).start()
    for b, k in ((ROWS_PER_W - 3, 2), (ROWS_PER_W - 2, 0), (ROWS_PER_W - 1, 1)):
        out_desc(b, k).wait()


def kernel(inputs, table):
    idx_flat = inputs.reshape(-1).astype(jnp.int32)
    pe = jnp.asarray(_pos_enc())
    return _emb(table, idx_flat, pe)


# speedup vs baseline: 1.0280x; 1.0043x over previous
import functools

import jax
import jax.numpy as jnp
import numpy as np
from jax import lax
from jax.experimental import pallas as pl
from jax.experimental.pallas import tpu as pltpu
from jax.experimental.pallas import tpu_sc as plsc

VOCAB = 100000
D = 128
S = 200
B = 1024

NC = 2
NS = 16
NW = NC * NS
ROWS_PER_W = B // NW
IDS_PER_W = ROWS_PER_W * S
NBUF = 3


def _pos_enc() -> np.ndarray:
    pos = np.arange(S, dtype=np.float32)[:, None]
    i = np.arange(D, dtype=np.float32)[None, :]
    angle_rates = 1.0 / np.power(10000.0, (2.0 * np.floor(i / 2.0)) / np.float32(D))
    angles = pos * angle_rates
    pe = np.zeros((S, D), dtype=np.float32)
    pe[:, 0::2] = np.sin(angles[:, 0::2])
    pe[:, 1::2] = np.cos(angles[:, 1::2])
    return pe


_MESH = plsc.VectorSubcoreMesh(core_axis_name="c", subcore_axis_name="s")


@functools.partial(
    pl.kernel,
    out_type=jax.ShapeDtypeStruct((B, S, D), jnp.float32),
    mesh=_MESH,
    scratch_types=[
        pltpu.VMEM((IDS_PER_W,), jnp.int32),
        pltpu.VMEM((S, D), jnp.float32),
        [pltpu.VMEM((S, D), jnp.float32)] * NBUF,
        [pltpu.SemaphoreType.DMA] * NBUF,
        [pltpu.SemaphoreType.DMA] * NBUF,
    ],
)
def _emb(table_hbm, idx_hbm, pe_hbm, out_hbm, idx_v, pe_v, bufs, gsems, osems):
    wid = lax.axis_index("s") * NC + lax.axis_index("c")
    icp = pltpu.make_async_copy(
        idx_hbm.at[pl.ds(wid * IDS_PER_W, IDS_PER_W)], idx_v, gsems[2])
    pcp = pltpu.make_async_copy(pe_hbm, pe_v, osems[2])
    icp.start()
    pcp.start()
    icp.wait()
    pcp.wait()

    def _gparts(b, k):
        return (
            (table_hbm.at[idx_v.at[pl.ds(b * S, 128)]],
             bufs[k].at[pl.ds(0, 128)], gsems[k]),
            (table_hbm.at[idx_v.at[pl.ds(b * S + 128, S - 128)]],
             bufs[k].at[pl.ds(128, S - 128)], gsems[k]),
        )

    def issue_gather(b, k):
        prefill_pe(k)
        for src, dst, sem in _gparts(b, k):
            pltpu.async_copy(src, dst, sem, add=True)

    def wait_gather(b, k):
        for src, dst, sem in _gparts(b, k):
            pltpu.make_async_copy(src, dst, sem).wait()

    def prefill_pe(k):
        buf = bufs[k]

        def cp_rows(t, c2):
            for u in range(2):
                i = 2 * t + u
                for j in range(D // 16):
                    sl = pl.ds(j * 16, 16)
                    buf[i, sl] = pe_v[i, sl]
            return c2

        lax.fori_loop(0, S // 2, cp_rows, 0)

    def out_desc(b, k):
        return pltpu.make_async_copy(
            bufs[k], out_hbm.at[wid * ROWS_PER_W + b], osems[k])

    issue_gather(0, 0)
    issue_gather(1, 1)

    def trio(p, carry):
        for j in range(NBUF):
            b = NBUF * p + j
            wait_gather(b, j)
            kn = (j + NBUF - 1) % NBUF

            def recycle():
                out_desc(b - 1, kn).wait()
                issue_gather(b + 2, kn)

            if j == 0:
                @pl.when(p > 0)
                def _():
                    recycle()

                @pl.when(p == 0)
                def _():
                    issue_gather(b + 2, kn)
            else:
                recycle()
            out_desc(b, j).start()
        return carry

    lax.fori_loop(0, (ROWS_PER_W - 2) // NBUF, trio, 0)
    for b, k in ((ROWS_PER_W - 2, 0), (ROWS_PER_W - 1, 1)):
        wait_gather(b, k)
        out_desc(b, k).start()
    for b, k in ((ROWS_PER_W - 3, 2), (ROWS_PER_W - 2, 0), (ROWS_PER_W - 1, 1)):
        out_desc(b, k).wait()


def kernel(inputs, table):
    idx_flat = inputs.reshape(-1).astype(jnp.int32)
    pe = jnp.asarray(_pos_enc())
    return _emb(table, idx_flat, pe)


# speedup vs baseline: 1.0321x; 1.0040x over previous
import functools

import jax
import jax.numpy as jnp
import numpy as np
from jax import lax
from jax.experimental import pallas as pl
from jax.experimental.pallas import tpu as pltpu
from jax.experimental.pallas import tpu_sc as plsc

VOCAB = 100000
D = 128
S = 200
B = 1024
N = B * S

NC = 2
NS = 16
NW = NC * NS
IDS_PER_W = N // NW
CHUNK = 40
NCHUNK = IDS_PER_W // CHUNK
NBUF = 10


def _pos_enc() -> np.ndarray:
    pos = np.arange(S, dtype=np.float32)[:, None]
    i = np.arange(D, dtype=np.float32)[None, :]
    angle_rates = 1.0 / np.power(10000.0, (2.0 * np.floor(i / 2.0)) / np.float32(D))
    angles = pos * angle_rates
    pe = np.zeros((S, D), dtype=np.float32)
    pe[:, 0::2] = np.sin(angles[:, 0::2])
    pe[:, 1::2] = np.cos(angles[:, 1::2])
    return pe


_MESH = plsc.VectorSubcoreMesh(core_axis_name="c", subcore_axis_name="s")


@functools.partial(
    pl.kernel,
    out_type=jax.ShapeDtypeStruct((N, D), jnp.float32),
    mesh=_MESH,
    scratch_types=[
        pltpu.VMEM((IDS_PER_W,), jnp.int32),
        pltpu.VMEM((S, D), jnp.float32),
        [pltpu.VMEM((CHUNK, D), jnp.float32)] * NBUF,
        [pltpu.SemaphoreType.DMA] * NBUF,
        [pltpu.SemaphoreType.DMA] * NBUF,
    ],
)
def _emb(table_hbm, idx_hbm, pe_hbm, out_hbm, idx_v, pe_v, bufs, gsems, osems):
    wid = lax.axis_index("s") * NC + lax.axis_index("c")
    base = wid * IDS_PER_W
    icp = pltpu.make_async_copy(idx_hbm.at[pl.ds(base, IDS_PER_W)], idx_v,
                                gsems[NBUF - 1])
    pcp = pltpu.make_async_copy(pe_hbm, pe_v, osems[NBUF - 1])
    icp.start()
    pcp.start()
    icp.wait()
    pcp.wait()

    def gather_desc(c, k):
        return pltpu.make_async_copy(
            table_hbm.at[idx_v.at[pl.ds(c * CHUNK, CHUNK)]], bufs[k], gsems[k])

    def out_desc(c, k):
        return pltpu.make_async_copy(
            bufs[k], out_hbm.at[pl.ds(base + c * CHUNK, CHUNK)], osems[k])

    def prefill_pe(k):
        buf = bufs[k]
        s0 = (k % (S // CHUNK)) * CHUNK

        def cp_rows(t, c2):
            for u in range(2):
                i = 2 * t + u
                for j in range(D // 16):
                    sl = pl.ds(j * 16, 16)
                    buf[i, sl] = pe_v[s0 + i, sl]
            return c2

        lax.fori_loop(0, CHUNK // 2, cp_rows, 0)

    def issue_gather(c, k):
        prefill_pe(k)
        src = table_hbm.at[idx_v.at[pl.ds(c * CHUNK, CHUNK)]]
        pltpu.async_copy(src, bufs[k], gsems[k], add=True)

    for k in range(NBUF - 1):
        issue_gather(k, k)

    def ring(p, carry):
        for j in range(NBUF):
            c = NBUF * p + j
            gather_desc(c, j).wait()
            out_desc(c, j).start()
            kn = (j + NBUF - 1) % NBUF

            def recycle():
                out_desc(c - 1, kn).wait()
                issue_gather(c + NBUF - 1, kn)

            if j == 0:
                @pl.when(p > 0)
                def _():
                    recycle()

                @pl.when(p == 0)
                def _():
                    issue_gather(c + NBUF - 1, kn)
            else:
                @pl.when(p < NCHUNK // NBUF - 1)
                def _():
                    recycle()
        return carry

    lax.fori_loop(0, NCHUNK // NBUF, ring, 0)
    for j in range(NBUF):
        out_desc(NCHUNK - NBUF + j, j).wait()


def kernel(inputs, table):
    idx_flat = inputs.reshape(-1).astype(jnp.int32)
    pe = jnp.asarray(_pos_enc())
    return _emb(table, idx_flat, pe).reshape(B, S, D)
